# baseline (speedup 1.0000x reference)
import functools

import jax
import jax.numpy as jnp
from jax import lax
from jax.experimental import pallas as pl
from jax.experimental.pallas import tpu as pltpu
from jax.experimental.pallas import tpu_sc as plsc

NT = 16
NC = 2
SEG = 128
BN = 1024


def _make_sc_gather(npad, npt, qidx, kcnt, grp):
    nseg = npt // SEG
    mesh = plsc.VectorSubcoreMesh(core_axis_name="c", subcore_axis_name="s")

    khalf = kcnt // NC

    @functools.partial(
        pl.kernel,
        out_type=jax.ShapeDtypeStruct((kcnt, 8, npad), jnp.float32),
        mesh=mesh,
        scratch_types=[
            pltpu.VMEM((npad,), jnp.float32),
            pltpu.VMEM((npad,), jnp.float32),
            pltpu.VMEM((npad,), jnp.float32),
            pltpu.VMEM((grp, SEG), jnp.int32),
            pltpu.VMEM((khalf, 3, npt), jnp.float32),
        ],
        compiler_params=pltpu.CompilerParams(needs_layout_passes=False),
    )
    def sc_gather(pos8_hbm, idx_hbm, pnb_hbm, p0_v, p1_v, p2_v, idx_v, out_v):
        c = lax.axis_index("c")
        s = lax.axis_index("s")
        pltpu.sync_copy(pos8_hbm.at[0], p0_v)
        pltpu.sync_copy(pos8_hbm.at[1], p1_v)
        pltpu.sync_copy(pos8_hbm.at[2], p2_v)
        off = pl.multiple_of((qidx * NC + c) * grp, 8)
        pltpu.sync_copy(idx_hbm.at[s, pl.ds(off, grp)], idx_v)

        def kbody(kk, carry):
            for seg in range(nseg):
                for t in range(SEG // 16):
                    off = seg * SEG + t * 16
                    i16 = idx_v[kk * nseg + seg, pl.ds(t * 16, 16)]
                    out_v[kk, 0, pl.ds(off, 16)] = plsc.load_gather(p0_v, [i16])
                    out_v[kk, 1, pl.ds(off, 16)] = plsc.load_gather(p1_v, [i16])
                    out_v[kk, 2, pl.ds(off, 16)] = plsc.load_gather(p2_v, [i16])
            return carry

        lax.fori_loop(0, khalf, kbody, 0)
        pltpu.sync_copy(out_v,
                        pnb_hbm.at[pl.ds(c * khalf, khalf), pl.ds(0, 3),
                                   pl.ds(s * npt, npt)])

    return sc_gather


def _prep_body(f_ref, w_ref, pos_ref, h_ref):
    p = lax.dot_general(f_ref[...], w_ref[...], (((0,), (0,)), ((), ())),
                        preferred_element_type=jnp.float32)
    g = p[:, 384:512]
    for d in range(3):
        g = g - pos_ref[d, :][:, None] * p[:, d * 128:(d + 1) * 128]
    h_ref[0] = p[:, 0:128]
    h_ref[1] = p[:, 128:256]
    h_ref[2] = p[:, 256:384]
    h_ref[3] = g


def _msg_body(nk, h_ref, pnb_ref, msg_ref):
    t = jnp.transpose(pnb_ref[...].reshape(nk * 8, -1), (1, 0))
    for j in range(nk):
        pnb_t = t[:, 8 * j:8 * j + 8]
        acc = h_ref[3]
        for d in range(3):
            acc = acc + pnb_t[:, d:d + 1] * h_ref[d]
        msg_ref[j] = acc


def _make_sc_scatter(npad, npt, qidx, kcnt, grp):
    nseg = npt // SEG
    khalf = kcnt // NC
    nunit = khalf * nseg
    mesh = plsc.VectorSubcoreMesh(core_axis_name="c", subcore_axis_name="s")

    @functools.partial(
        pl.kernel,
        out_type=jax.ShapeDtypeStruct((NC, npad, 128), jnp.float32),
        mesh=mesh,
        scratch_types=[
            pltpu.VMEM((SEG, 128), jnp.float32),
            pltpu.VMEM((SEG, 128), jnp.float32),
            pltpu.VMEM((grp, SEG), jnp.int32),
            pltpu.VMEM_SHARED((npad, 128), jnp.float32),
            pltpu.SemaphoreType.DMA,
            pltpu.SemaphoreType.DMA,
        ],
    )
    def sc_scatter(msg_hbm, idx_hbm, z_hbm, s2_hbm, b0, b1, idx_v, shared,
                   sem0, sem1):
        c = lax.axis_index("c")
        s = lax.axis_index("s")
        base = s * npt
        off = pl.multiple_of((qidx * NC + c) * grp, 8)
        pltpu.sync_copy(idx_hbm.at[s, pl.ds(off, grp)], idx_v)
        pltpu.sync_copy(z_hbm, shared.at[pl.ds(base, npt)])
        plsc.subcore_barrier()

        def _src(u):
            kk = u // nseg
            seg = u - kk * nseg
            return msg_hbm.at[c * khalf + kk, pl.ds(base + seg * SEG, SEG)]

        pltpu.async_copy(_src(0), b0, sem0)

        def ubody(i, carry):
            u0 = 2 * i
            pltpu.async_copy(_src(u0 + 1), b1, sem1)
            pltpu.make_async_copy(_src(u0), b0, sem0).wait()
            pltpu.sync_copy(b0, shared.at[idx_v.at[u0]], add=True)

            @pl.when(i < nunit // 2 - 1)
            def _():
                pltpu.async_copy(_src(u0 + 2), b0, sem0)

            pltpu.make_async_copy(_src(u0 + 1), b1, sem1).wait()
            pltpu.sync_copy(b1, shared.at[idx_v.at[u0 + 1]], add=True)
            return carry

        lax.fori_loop(0, nunit // 2, ubody, 0)
        plsc.subcore_barrier()
        pltpu.sync_copy(shared.at[pl.ds(base, npt)],
                        s2_hbm.at[c, pl.ds(base, npt)])

    return sc_scatter


def _comb_body(*refs):
    s_refs, b_ref, o_ref = refs[:-2], refs[-2], refs[-1]
    acc = b_ref[0, :][None, :]
    for s_ref in s_refs:
        acc = acc + s_ref[0] + s_ref[1]
    o_ref[...] = acc.T


def kernel(features, weight_theta, weight_bias, bias, neighborhood, positions):
    b, din, n = features.shape
    k = neighborhood.shape[1]
    dout = weight_theta.shape[-1]
    npt = ((n + NT * SEG - 1) // (NT * SEG)) * SEG
    npad = npt * NT
    nseg = npt // SEG

    f_pad = jnp.pad(features[0], ((0, 0), (0, npad - n)))
    pos8 = jnp.pad(positions[0], ((0, 5), (0, npad - n)))
    wcat = jnp.concatenate(
        [weight_theta[0], weight_theta[1], weight_theta[2], weight_bias], axis=1)
    bias_pad = jnp.pad(bias[None, :], ((0, 7), (0, 0)))
    nb_pad = jnp.pad(neighborhood[0], ((0, 0), (0, npad - n)))
    idx = nb_pad.reshape(k, NT, nseg, SEG).transpose(1, 0, 2, 3).reshape(
        NT, k * nseg, SEG)
    nq = 4
    kh = k // nq
    grows = (kh // NC) * nseg
    grp = ((grows + 7) // 8) * 8
    idx = idx.reshape(NT, nq * NC, grows, SEG)
    idx = jnp.pad(idx, ((0, 0), (0, 0), (0, grp - grows), (0, 0)))
    idx = idx.reshape(NT, nq * NC * grp, SEG)
    z = jnp.zeros((npt, 128), jnp.float32)

    h4 = pl.pallas_call(
        _prep_body,
        grid=(npad // BN,),
        in_specs=[
            pl.BlockSpec((din, BN), lambda i: (0, i)),
            pl.BlockSpec((din, 4 * dout), lambda i: (0, 0)),
            pl.BlockSpec((8, BN), lambda i: (0, i)),
        ],
        out_specs=pl.BlockSpec((4, BN, dout), lambda i: (0, i, 0)),
        out_shape=jax.ShapeDtypeStruct((4, npad, dout), jnp.float32),
    )(f_pad, wcat, pos8)

    s2s = []
    for q in range(nq):
        posnb = _make_sc_gather(npad, npt, q, kh, grp)(pos8, idx)
        msg = pl.pallas_call(
            functools.partial(_msg_body, kh),
            grid=(npad // BN,),
            in_specs=[
                pl.BlockSpec((4, BN, dout), lambda i: (0, i, 0)),
                pl.BlockSpec((kh, 8, BN), lambda i: (0, 0, i)),
            ],
            out_specs=pl.BlockSpec((kh, BN, dout), lambda i: (0, i, 0)),
            out_shape=jax.ShapeDtypeStruct((kh, npad, dout), jnp.float32),
        )(h4, posnb)
        s2s.append(_make_sc_scatter(npad, npt, q, kh, grp)(msg, idx, z))

    o_t = pl.pallas_call(
        _comb_body,
        grid=(npad // BN,),
        in_specs=[pl.BlockSpec((NC, BN, dout), lambda i: (0, i, 0))] * nq
        + [pl.BlockSpec((8, dout), lambda i: (0, 0))],
        out_specs=pl.BlockSpec((dout, BN), lambda i: (0, i)),
        out_shape=jax.ShapeDtypeStruct((dout, npad), jnp.float32),
    )(*s2s, bias_pad)

    return o_t[None, :, :n]


# speedup vs baseline: 1.1451x; 1.1451x over previous
import functools

import jax
import jax.numpy as jnp
from jax import lax
from jax.experimental import pallas as pl
from jax.experimental.pallas import tpu as pltpu
from jax.experimental.pallas import tpu_sc as plsc

NT = 16
NC = 2
SEG = 128
BN = 1024


def _make_sc_gather(npad, npt, qidx, kcnt, grp):
    nseg = npt // SEG
    mesh = plsc.VectorSubcoreMesh(core_axis_name="c", subcore_axis_name="s")

    khalf = kcnt // NC

    @functools.partial(
        pl.kernel,
        out_type=jax.ShapeDtypeStruct((kcnt, 8, npad), jnp.float32),
        mesh=mesh,
        scratch_types=[
            pltpu.VMEM((npad,), jnp.float32),
            pltpu.VMEM((npad,), jnp.float32),
            pltpu.VMEM((npad,), jnp.float32),
            pltpu.VMEM((grp, SEG), jnp.int32),
            pltpu.VMEM((khalf, 3, npt), jnp.float32),
        ],
        compiler_params=pltpu.CompilerParams(needs_layout_passes=False),
    )
    def sc_gather(pos8_hbm, idx_hbm, pnb_hbm, p0_v, p1_v, p2_v, idx_v, out_v):
        c = lax.axis_index("c")
        s = lax.axis_index("s")
        pltpu.sync_copy(pos8_hbm.at[0], p0_v)
        pltpu.sync_copy(pos8_hbm.at[1], p1_v)
        pltpu.sync_copy(pos8_hbm.at[2], p2_v)
        off = pl.multiple_of((qidx * NC + c) * grp, 8)
        pltpu.sync_copy(idx_hbm.at[s, pl.ds(off, grp)], idx_v)

        def kbody(kk, carry):
            for seg in range(nseg):
                for t in range(SEG // 16):
                    off = seg * SEG + t * 16
                    i16 = idx_v[kk * nseg + seg, pl.ds(t * 16, 16)]
                    out_v[kk, 0, pl.ds(off, 16)] = plsc.load_gather(p0_v, [i16])
                    out_v[kk, 1, pl.ds(off, 16)] = plsc.load_gather(p1_v, [i16])
                    out_v[kk, 2, pl.ds(off, 16)] = plsc.load_gather(p2_v, [i16])
            return carry

        lax.fori_loop(0, khalf, kbody, 0)
        pltpu.sync_copy(out_v,
                        pnb_hbm.at[pl.ds(c * khalf, khalf), pl.ds(0, 3),
                                   pl.ds(s * npt, npt)])

    return sc_gather


def _prep_body(f_ref, w_ref, pos_ref, h_ref):
    p = lax.dot_general(f_ref[...], w_ref[...], (((0,), (0,)), ((), ())),
                        preferred_element_type=jnp.float32)
    g = p[:, 384:512]
    for d in range(3):
        g = g - pos_ref[d, :][:, None] * p[:, d * 128:(d + 1) * 128]
    h_ref[0] = p[:, 0:128]
    h_ref[1] = p[:, 128:256]
    h_ref[2] = p[:, 256:384]
    h_ref[3] = g


def _msg_body(nk, h_ref, pnb_ref, msg_ref):
    t = jnp.transpose(pnb_ref[...].reshape(nk * 8, -1), (1, 0))
    for j in range(nk):
        pnb_t = t[:, 8 * j:8 * j + 8]
        acc = h_ref[3]
        for d in range(3):
            acc = acc + pnb_t[:, d:d + 1] * h_ref[d]
        msg_ref[j] = acc


def _make_sc_scatter(npad, npt, qidx, kcnt, grp):
    nseg = npt // SEG
    khalf = kcnt // NC
    nunit = khalf * nseg
    mesh = plsc.VectorSubcoreMesh(core_axis_name="c", subcore_axis_name="s")

    @functools.partial(
        pl.kernel,
        out_type=jax.ShapeDtypeStruct((NC, npad, 128), jnp.float32),
        mesh=mesh,
        scratch_types=[
            pltpu.VMEM((SEG, 128), jnp.float32),
            pltpu.VMEM((SEG, 128), jnp.float32),
            pltpu.VMEM((grp, SEG), jnp.int32),
            pltpu.VMEM_SHARED((npad, 128), jnp.float32),
            pltpu.SemaphoreType.DMA,
            pltpu.SemaphoreType.DMA,
        ],
    )
    def sc_scatter(msg_hbm, idx_hbm, z_hbm, s2_hbm, b0, b1, idx_v, shared,
                   sem0, sem1):
        c = lax.axis_index("c")
        s = lax.axis_index("s")
        base = s * npt
        off = pl.multiple_of((qidx * NC + c) * grp, 8)
        pltpu.sync_copy(idx_hbm.at[s, pl.ds(off, grp)], idx_v)
        pltpu.sync_copy(z_hbm, shared.at[pl.ds(base, npt)])
        plsc.subcore_barrier()

        def _src(u):
            kk = u // nseg
            seg = u - kk * nseg
            return msg_hbm.at[c * khalf + kk, pl.ds(base + seg * SEG, SEG)]

        pltpu.async_copy(_src(0), b0, sem0)

        def ubody(i, carry):
            u0 = 2 * i
            pltpu.async_copy(_src(u0 + 1), b1, sem1)
            pltpu.make_async_copy(_src(u0), b0, sem0).wait()
            pltpu.sync_copy(b0, shared.at[idx_v.at[u0]], add=True)

            @pl.when(i < nunit // 2 - 1)
            def _():
                pltpu.async_copy(_src(u0 + 2), b0, sem0)

            pltpu.make_async_copy(_src(u0 + 1), b1, sem1).wait()
            pltpu.sync_copy(b1, shared.at[idx_v.at[u0 + 1]], add=True)
            return carry

        lax.fori_loop(0, nunit // 2, ubody, 0)
        plsc.subcore_barrier()
        pltpu.sync_copy(shared.at[pl.ds(base, npt)],
                        s2_hbm.at[c, pl.ds(base, npt)])

    return sc_scatter


def _comb_body(*refs):
    s_refs, b_ref, o_ref = refs[:-2], refs[-2], refs[-1]
    acc = b_ref[0, :][None, :]
    for s_ref in s_refs:
        acc = acc + s_ref[0] + s_ref[1]
    o_ref[...] = acc.T


def kernel(features, weight_theta, weight_bias, bias, neighborhood, positions):
    b, din, n = features.shape
    k = neighborhood.shape[1]
    dout = weight_theta.shape[-1]
    npt = ((n + NT * SEG - 1) // (NT * SEG)) * SEG
    npad = npt * NT
    nseg = npt // SEG

    f_pad = jnp.pad(features[0], ((0, 0), (0, npad - n)))
    pos8 = jnp.pad(positions[0], ((0, 5), (0, npad - n)))
    wcat = jnp.concatenate(
        [weight_theta[0], weight_theta[1], weight_theta[2], weight_bias], axis=1)
    bias_pad = jnp.pad(bias[None, :], ((0, 7), (0, 0)))
    nb_pad = jnp.pad(neighborhood[0], ((0, 0), (0, npad - n)))
    idx = nb_pad.reshape(k, NT, nseg, SEG).transpose(1, 0, 2, 3).reshape(
        NT, k * nseg, SEG)
    nq = 2
    kh = k // nq
    grows = (kh // NC) * nseg
    grp = ((grows + 7) // 8) * 8
    idx = idx.reshape(NT, nq * NC, grows, SEG)
    idx = jnp.pad(idx, ((0, 0), (0, 0), (0, grp - grows), (0, 0)))
    idx = idx.reshape(NT, nq * NC * grp, SEG)
    z = jnp.zeros((npt, 128), jnp.float32)

    h4 = pl.pallas_call(
        _prep_body,
        grid=(npad // BN,),
        in_specs=[
            pl.BlockSpec((din, BN), lambda i: (0, i)),
            pl.BlockSpec((din, 4 * dout), lambda i: (0, 0)),
            pl.BlockSpec((8, BN), lambda i: (0, i)),
        ],
        out_specs=pl.BlockSpec((4, BN, dout), lambda i: (0, i, 0)),
        out_shape=jax.ShapeDtypeStruct((4, npad, dout), jnp.float32),
    )(f_pad, wcat, pos8)

    s2s = []
    for q in range(nq):
        posnb = _make_sc_gather(npad, npt, q, kh, grp)(pos8, idx)
        msg = pl.pallas_call(
            functools.partial(_msg_body, kh),
            grid=(npad // BN,),
            in_specs=[
                pl.BlockSpec((4, BN, dout), lambda i: (0, i, 0)),
                pl.BlockSpec((kh, 8, BN), lambda i: (0, 0, i)),
            ],
            out_specs=pl.BlockSpec((kh, BN, dout), lambda i: (0, i, 0)),
            out_shape=jax.ShapeDtypeStruct((kh, npad, dout), jnp.float32),
        )(h4, posnb)
        s2s.append(_make_sc_scatter(npad, npt, q, kh, grp)(msg, idx, z))

    o_t = pl.pallas_call(
        _comb_body,
        grid=(npad // BN,),
        in_specs=[pl.BlockSpec((NC, BN, dout), lambda i: (0, i, 0))] * nq
        + [pl.BlockSpec((8, dout), lambda i: (0, 0))],
        out_specs=pl.BlockSpec((dout, BN), lambda i: (0, i)),
        out_shape=jax.ShapeDtypeStruct((dout, npad), jnp.float32),
    )(*s2s, bias_pad)

    return o_t[None, :, :n]
